# Initial kernel scaffold; baseline (speedup 1.0000x reference)
import jax
import jax.numpy as jnp
from jax.experimental import pallas as pl


def kernel(embeddings, all_spans, W, b):
    raise NotImplementedError("write your pallas kernel here")


# speedup vs baseline: 1.2160x; 1.2160x over previous
import functools

import jax
import jax.numpy as jnp
from jax import lax
from jax.experimental import pallas as pl
from jax.experimental.pallas import tpu as pltpu
from jax.experimental.pallas import tpu_sc as plsc

SEQ = 4096
HID = 768
NSPANS = 2048
BLK = 128
NBLK = SEQ // BLK
SPAD = SEQ + 8
OPAD = NBLK + 8

NC = 2
NS = 16
NW = NC * NS
SP_PER_W = NSPANS // NW
GRP = 16
NGRP = SP_PER_W // GRP


def _prefix_body(emb_ref, w_ref, b_ref, lix_ref, offx_ref, lie_ref, offe_ref):
    emb = emb_ref[...]
    scores = jnp.sum(emb * w_ref[...], axis=1, keepdims=True) + b_ref[0, 0]
    gmax = jnp.max(scores)
    e = jnp.exp(scores - gmax)

    row = lax.broadcasted_iota(jnp.int32, (BLK, BLK), 0)
    col = lax.broadcasted_iota(jnp.int32, (BLK, BLK), 1)
    tinc = (row >= col).astype(jnp.float32)
    rowb = lax.broadcasted_iota(jnp.int32, (NBLK, NBLK), 0)
    colb = lax.broadcasted_iota(jnp.int32, (NBLK, NBLK), 1)
    texc = (rowb > colb).astype(jnp.float32)

    sx_rows = []
    se_rows = []
    for k in range(NBLK):
        sl = slice(k * BLK, (k + 1) * BLK)
        ek = e[sl]
        xk = emb[sl] * ek
        lixk = jnp.dot(tinc, xk, preferred_element_type=jnp.float32,
                       precision=lax.Precision.HIGHEST)
        liek = jnp.dot(tinc, ek, preferred_element_type=jnp.float32,
                       precision=lax.Precision.HIGHEST)
        lix_ref[sl, :] = lixk
        lie_ref[sl, :] = liek
        sx_rows.append(lixk[BLK - 1:BLK, :])
        se_rows.append(liek[BLK - 1:BLK, :])
    lix_ref[SEQ:SPAD, :] = jnp.zeros((SPAD - SEQ, HID), jnp.float32)
    lie_ref[SEQ:SPAD, :] = jnp.zeros((SPAD - SEQ, 1), jnp.float32)

    sx = jnp.concatenate(sx_rows, axis=0)
    se = jnp.concatenate(se_rows, axis=0)
    offx_ref[0:NBLK, :] = jnp.dot(texc, sx, preferred_element_type=jnp.float32,
                                  precision=lax.Precision.HIGHEST)
    offe_ref[0:NBLK, :] = jnp.dot(texc, se, preferred_element_type=jnp.float32,
                                  precision=lax.Precision.HIGHEST)
    offx_ref[NBLK:OPAD, :] = jnp.zeros((OPAD - NBLK, HID), jnp.float32)
    offe_ref[NBLK:OPAD, :] = jnp.zeros((OPAD - NBLK, 1), jnp.float32)


def _prefix_stage(emb, w, b):
    return pl.pallas_call(
        _prefix_body,
        out_shape=(
            jax.ShapeDtypeStruct((SPAD, HID), jnp.float32),
            jax.ShapeDtypeStruct((OPAD, HID), jnp.float32),
            jax.ShapeDtypeStruct((SPAD, 1), jnp.float32),
            jax.ShapeDtypeStruct((OPAD, 1), jnp.float32),
        ),
    )(emb, w, b)


def _span_body(lix_hbm, offx_hbm, lie_hbm, offe_hbm, emb_hbm, spans_hbm,
               out_hbm, spans_v, lie_v, offe_v,
               eidx, sidx, spidx, beidx, bspidx,
               ge, gs, oe, os_, ems, eme, attn_v, sem):
    wid = lax.axis_index("s") * NC + lax.axis_index("c")
    pltpu.sync_copy(spans_hbm.at[pl.ds(wid * (2 * SP_PER_W), 2 * SP_PER_W)],
                    spans_v)
    pltpu.sync_copy(lie_hbm, lie_v)
    pltpu.sync_copy(offe_hbm, offe_v)

    lane = lax.iota(jnp.int32, GRP)
    for g in range(NGRP):
        base = wid * SP_PER_W + g * GRP
        gi = (g * GRP + lane) * 2
        s_vec = plsc.load_gather(spans_v, [gi])
        e_vec = plsc.load_gather(spans_v, [gi + 1])
        s_is0 = s_vec == 0
        sp_vec = jnp.where(s_is0, SEQ, s_vec - 1)
        be_vec = lax.shift_right_logical(e_vec, 7)
        bsp_vec = jnp.where(s_is0, NBLK,
                            lax.shift_right_logical(s_vec - 1, 7))
        eidx[...] = e_vec
        sidx[...] = s_vec
        spidx[...] = sp_vec
        beidx[...] = be_vec
        bspidx[...] = bsp_vec

        copies = [
            pltpu.async_copy(lix_hbm.at[eidx], ge, sem),
            pltpu.async_copy(lix_hbm.at[spidx], gs, sem),
            pltpu.async_copy(offx_hbm.at[beidx], oe, sem),
            pltpu.async_copy(offx_hbm.at[bspidx], os_, sem),
            pltpu.async_copy(emb_hbm.at[sidx], ems, sem),
            pltpu.async_copy(emb_hbm.at[eidx], eme, sem),
        ]
        den = (plsc.load_gather(offe_v, [be_vec])
               + plsc.load_gather(lie_v, [e_vec])
               - plsc.load_gather(offe_v, [bsp_vec])
               - plsc.load_gather(lie_v, [sp_vec]))
        inv_vec = 1.0 / den
        inv = [inv_vec[j] for j in range(GRP)]
        for c in copies:
            c.wait()

        def chunk(c, carry):
            o = c * 16
            for j in range(GRP):
                num = ((oe[j, pl.ds(o, 16)] - os_[j, pl.ds(o, 16)])
                       + (ge[j, pl.ds(o, 16)] - gs[j, pl.ds(o, 16)]))
                attn_v[j, pl.ds(o, 16)] = num * inv[j]
            return carry

        lax.fori_loop(0, HID // 16, chunk, 0)

        pltpu.sync_copy(ems, out_hbm.at[pl.ds(base, GRP), pl.ds(0, HID)])
        pltpu.sync_copy(eme, out_hbm.at[pl.ds(base, GRP), pl.ds(HID, HID)])
        pltpu.sync_copy(attn_v,
                        out_hbm.at[pl.ds(base, GRP), pl.ds(2 * HID, HID)])


@functools.cache
def _make_span_stage():
    return functools.partial(
        pl.kernel,
        out_type=jax.ShapeDtypeStruct((NSPANS, 3 * HID), jnp.float32),
        mesh=plsc.VectorSubcoreMesh(core_axis_name="c", subcore_axis_name="s"),
        compiler_params=pltpu.CompilerParams(needs_layout_passes=False),
        scratch_types=[
            pltpu.VMEM((2 * SP_PER_W,), jnp.int32),
            pltpu.VMEM((SPAD,), jnp.float32),
            pltpu.VMEM((OPAD,), jnp.float32),
            pltpu.VMEM((GRP,), jnp.int32),
            pltpu.VMEM((GRP,), jnp.int32),
            pltpu.VMEM((GRP,), jnp.int32),
            pltpu.VMEM((GRP,), jnp.int32),
            pltpu.VMEM((GRP,), jnp.int32),
            pltpu.VMEM((GRP, HID), jnp.float32),
            pltpu.VMEM((GRP, HID), jnp.float32),
            pltpu.VMEM((GRP, HID), jnp.float32),
            pltpu.VMEM((GRP, HID), jnp.float32),
            pltpu.VMEM((GRP, HID), jnp.float32),
            pltpu.VMEM((GRP, HID), jnp.float32),
            pltpu.VMEM((GRP, HID), jnp.float32),
            pltpu.SemaphoreType.DMA,
        ],
    )(_span_body)


@jax.jit
def kernel(embeddings, all_spans, W, b):
    emb = embeddings[0]
    w2 = W.reshape(1, HID)
    b2 = b.reshape(1, 1)
    lix, offx, lie, offe = _prefix_stage(emb, w2, b2)
    spans_flat = all_spans.reshape(2 * NSPANS).astype(jnp.int32)
    return _make_span_stage()(lix, offx, lie.reshape(SPAD),
                              offe.reshape(OPAD), emb, spans_flat)


# speedup vs baseline: 1.4840x; 1.2204x over previous
import functools

import jax
import jax.numpy as jnp
from jax import lax
from jax.experimental import pallas as pl
from jax.experimental.pallas import tpu as pltpu
from jax.experimental.pallas import tpu_sc as plsc

SEQ = 4096
HID = 768
NSPANS = 2048
BLK = 128
NBLK = SEQ // BLK
SPAD = SEQ + 8
OPAD = NBLK + 8

NC = 2
NS = 16
NW = NC * NS
SP_PER_W = NSPANS // NW
GRP = 8
NGRP = SP_PER_W // GRP
NBUF = 3
NCH = HID // 16


def _prefix_body(emb_ref, w_ref, b_ref, lix_ref, offx_ref, lie_ref, offe_ref):
    emb = emb_ref[...]
    scores = jnp.sum(emb * w_ref[...], axis=1, keepdims=True) + b_ref[0, 0]
    gmax = jnp.max(scores)
    e = jnp.exp(scores - gmax)

    row = lax.broadcasted_iota(jnp.int32, (BLK, BLK), 0)
    col = lax.broadcasted_iota(jnp.int32, (BLK, BLK), 1)
    tinc = (row >= col).astype(jnp.float32)
    rowb = lax.broadcasted_iota(jnp.int32, (NBLK, NBLK), 0)
    colb = lax.broadcasted_iota(jnp.int32, (NBLK, NBLK), 1)
    texc = (rowb > colb).astype(jnp.float32)

    sx_rows = []
    se_rows = []
    for k in range(NBLK):
        sl = slice(k * BLK, (k + 1) * BLK)
        ek = e[sl]
        xk = emb[sl] * ek
        lixk = jnp.dot(tinc, xk, preferred_element_type=jnp.float32)
        liek = jnp.dot(tinc, ek, preferred_element_type=jnp.float32,
                       precision=lax.Precision.HIGHEST)
        lix_ref[sl, :] = lixk
        lie_ref[k:k + 1, :] = jnp.transpose(liek)
        sx_rows.append(lixk[BLK - 1:BLK, :])
        se_rows.append(liek[BLK - 1:BLK, :])
    lix_ref[SEQ:SPAD, :] = jnp.zeros((SPAD - SEQ, HID), jnp.float32)
    lie_ref[NBLK:OPAD, :] = jnp.zeros((OPAD - NBLK, BLK), jnp.float32)

    sx = jnp.concatenate(sx_rows, axis=0)
    se = jnp.concatenate(se_rows, axis=0)
    offx_ref[0:NBLK, :] = jnp.dot(texc, sx, preferred_element_type=jnp.float32,
                                  precision=lax.Precision.HIGHEST)
    offe_cols = jnp.dot(texc, se, preferred_element_type=jnp.float32,
                        precision=lax.Precision.HIGHEST)
    offx_ref[NBLK:OPAD, :] = jnp.zeros((OPAD - NBLK, HID), jnp.float32)
    offe_ref[...] = jnp.zeros((8, BLK), jnp.float32)
    offe_ref[0:1, 0:NBLK] = jnp.transpose(offe_cols)


def _prefix_stage(emb, w, b):
    return pl.pallas_call(
        _prefix_body,
        out_shape=(
            jax.ShapeDtypeStruct((SPAD, HID), jnp.float32),
            jax.ShapeDtypeStruct((OPAD, HID), jnp.float32),
            jax.ShapeDtypeStruct((OPAD, BLK), jnp.float32),
            jax.ShapeDtypeStruct((8, BLK), jnp.float32),
        ),
    )(emb, w, b)


def _span_body(lix_hbm, offx_hbm, lie_hbm, offe_hbm, emb_hbm, spans_hbm,
               out_hbm, spans_v, lie_v, offe_v, offx_v,
               ge0, gs0, ems0, eme0, ge1, gs1, ems1, eme1,
               ge2, gs2, ems2, eme2,
               ei0, si0, pi0, ei1, si1, pi1, ei2, si2, pi2,
               gsem0, gsem1, gsem2, osem0, osem1, osem2):
    ge = [ge0, ge1, ge2]
    gs = [gs0, gs1, gs2]
    ems = [ems0, ems1, ems2]
    eme = [eme0, eme1, eme2]
    eidx = [ei0, ei1, ei2]
    sidx = [si0, si1, si2]
    spidx = [pi0, pi1, pi2]
    gsem = [gsem0, gsem1, gsem2]
    osem = [osem0, osem1, osem2]

    wid = lax.axis_index("s") * NC + lax.axis_index("c")
    pltpu.sync_copy(spans_hbm.at[pl.ds(wid * SP_PER_W, SP_PER_W), :], spans_v)
    pltpu.sync_copy(lie_hbm, lie_v)
    pltpu.sync_copy(offe_hbm, offe_v)
    pltpu.sync_copy(offx_hbm, offx_v)

    lane = lax.iota(jnp.int32, 16)
    zz = jnp.zeros((16,), jnp.int32)
    zo = jnp.ones((16,), jnp.int32)

    def span_vecs(t):
        gidx = jnp.minimum(t * GRP + lane, SP_PER_W - 1)
        s_vec = plsc.load_gather(spans_v, [gidx, zz])
        e_vec = plsc.load_gather(spans_v, [gidx, zo])
        s_is0 = s_vec == 0
        sp_vec = jnp.where(s_is0, SEQ, s_vec - 1)
        be_vec = lax.shift_right_logical(e_vec, 7)
        bsp_vec = jnp.where(s_is0, NBLK,
                            lax.shift_right_logical(s_vec - 1, 7))
        return s_vec, e_vec, sp_vec, be_vec, bsp_vec

    def issue_gathers(t, k):
        s_vec, e_vec, sp_vec, _, _ = span_vecs(t)
        eidx[k][...] = e_vec
        sidx[k][...] = s_vec
        spidx[k][...] = sp_vec
        ei = eidx[k].at[pl.ds(0, GRP)]
        si = sidx[k].at[pl.ds(0, GRP)]
        pi = spidx[k].at[pl.ds(0, GRP)]
        return [
            pltpu.async_copy(lix_hbm.at[ei], ge[k], gsem[k]),
            pltpu.async_copy(lix_hbm.at[pi], gs[k], gsem[k]),
            pltpu.async_copy(emb_hbm.at[si], ems[k], gsem[k]),
            pltpu.async_copy(emb_hbm.at[ei], eme[k], gsem[k]),
        ]

    pend_g = {0: issue_gathers(0, 0)}
    pend_o = {}

    for g in range(NGRP):
        k = g % NBUF
        kn = (g + 1) % NBUF
        if g + 1 < NGRP:
            for c in pend_o.pop(kn, ()):
                c.wait()
            pend_g[kn] = issue_gathers(g + 1, kn)
        for c in pend_g.pop(k):
            c.wait()

        _, e_vec, sp_vec, be_vec, bsp_vec = span_vecs(g)
        c127 = jnp.full((16,), 127, jnp.int32)
        den = (plsc.load_gather(offe_v, [zz, be_vec])
               + plsc.load_gather(lie_v, [lax.shift_right_logical(e_vec, 7),
                                          e_vec & c127])
               - plsc.load_gather(offe_v, [zz, bsp_vec])
               - plsc.load_gather(lie_v, [lax.shift_right_logical(sp_vec, 7),
                                          sp_vec & c127]))
        inv_vec = 1.0 / den
        inv = [inv_vec[j] for j in range(GRP)]
        bej = [be_vec[j] for j in range(GRP)]
        bspj = [bsp_vec[j] for j in range(GRP)]

        gek, gsk = ge[k], gs[k]

        def chunk(c, carry):
            o = c * 16
            col = o + lane
            for j in range(GRP):
                oe = plsc.load_gather(
                    offx_v, [jnp.broadcast_to(bej[j], (16,)), col])
                os_ = plsc.load_gather(
                    offx_v, [jnp.broadcast_to(bspj[j], (16,)), col])
                num = ((oe - os_)
                       + (gek[j, pl.ds(o, 16)] - gsk[j, pl.ds(o, 16)]))
                gsk[j, pl.ds(o, 16)] = num * inv[j]
            return carry

        lax.fori_loop(0, NCH, chunk, 0)

        base = wid * SP_PER_W + g * GRP
        pend_o[k] = [
            pltpu.async_copy(ems[k],
                             out_hbm.at[pl.ds(base, GRP), pl.ds(0, HID)],
                             osem[k]),
            pltpu.async_copy(eme[k],
                             out_hbm.at[pl.ds(base, GRP), pl.ds(HID, HID)],
                             osem[k]),
            pltpu.async_copy(gs[k],
                             out_hbm.at[pl.ds(base, GRP), pl.ds(2 * HID, HID)],
                             osem[k]),
        ]

    for k in list(pend_o):
        for c in pend_o.pop(k):
            c.wait()


@functools.cache
def _make_span_stage():
    row_bufs = []
    for _ in range(NBUF):
        row_bufs += [pltpu.VMEM((GRP, HID), jnp.float32)] * 4
    idx_bufs = [pltpu.VMEM((16,), jnp.int32)] * (3 * NBUF)
    sems = [pltpu.SemaphoreType.DMA] * (2 * NBUF)
    return functools.partial(
        pl.kernel,
        out_type=jax.ShapeDtypeStruct((NSPANS, 3 * HID), jnp.float32),
        mesh=plsc.VectorSubcoreMesh(core_axis_name="c", subcore_axis_name="s"),
        compiler_params=pltpu.CompilerParams(needs_layout_passes=False),
        scratch_types=[
            pltpu.VMEM((SP_PER_W, 2), jnp.int32),
            pltpu.VMEM((OPAD, BLK), jnp.float32),
            pltpu.VMEM((8, BLK), jnp.float32),
            pltpu.VMEM((OPAD, HID), jnp.float32),
        ] + row_bufs + idx_bufs + sems,
    )(_span_body)


@jax.jit
def kernel(embeddings, all_spans, W, b):
    emb = embeddings[0]
    w2 = W.reshape(1, HID)
    b2 = b.reshape(1, 1)
    lix, offx, lie, offe = _prefix_stage(emb, w2, b2)
    return _make_span_stage()(lix, offx, lie, offe, emb,
                              all_spans.astype(jnp.int32))


# speedup vs baseline: 1.8588x; 1.2525x over previous
import functools

import jax
import jax.numpy as jnp
from jax import lax
from jax.experimental import pallas as pl
from jax.experimental.pallas import tpu as pltpu
from jax.experimental.pallas import tpu_sc as plsc

SEQ = 4096
HID = 768
NSPANS = 2048
BLK = 128
NBLK = SEQ // BLK
SPAD = SEQ + 8
OPAD = NBLK + 8

NC = 2
NS = 16
NW = NC * NS
SP_PER_W = NSPANS // NW
GRP = 8
NGRP = SP_PER_W // GRP
NBUF = 3
NCH = HID // 16


def _prefix_body(emb_ref, w_ref, b_ref, lix_ref, lie_ref, offe_ref):
    emb = emb_ref[...]
    scores = jnp.sum(emb * w_ref[...], axis=1, keepdims=True) + b_ref[0, 0]
    gmax = jnp.max(scores)
    e = jnp.exp(scores - gmax)

    row = lax.broadcasted_iota(jnp.int32, (BLK, BLK), 0)
    col = lax.broadcasted_iota(jnp.int32, (BLK, BLK), 1)
    tinc = (row >= col).astype(jnp.float32)
    rowb = lax.broadcasted_iota(jnp.int32, (NBLK, NBLK), 0)
    colb = lax.broadcasted_iota(jnp.int32, (NBLK, NBLK), 1)
    texc = (rowb > colb).astype(jnp.float32)

    sx_rows = []
    se_rows = []
    for k in range(NBLK):
        sl = slice(k * BLK, (k + 1) * BLK)
        ek = e[sl]
        xk = emb[sl] * ek
        lixk = jnp.dot(tinc, xk, preferred_element_type=jnp.float32)
        liek = jnp.dot(tinc, ek, preferred_element_type=jnp.float32,
                       precision=lax.Precision.HIGHEST)
        lix_ref[sl, :] = lixk
        lie_ref[k:k + 1, :] = jnp.transpose(liek)
        sx_rows.append(lixk[BLK - 1:BLK, :])
        se_rows.append(liek[BLK - 1:BLK, :])
    lix_ref[SEQ:SPAD, :] = jnp.zeros((SPAD - SEQ, HID), jnp.float32)
    lie_ref[NBLK:OPAD, :] = jnp.zeros((OPAD - NBLK, BLK), jnp.float32)

    sx = jnp.concatenate(sx_rows, axis=0)
    se = jnp.concatenate(se_rows, axis=0)
    offx = jnp.dot(texc, sx, preferred_element_type=jnp.float32,
                   precision=lax.Precision.HIGHEST)
    for k in range(NBLK):
        sl = slice(k * BLK, (k + 1) * BLK)
        lix_ref[sl, :] = lix_ref[sl, :] + offx[k:k + 1, :]
    offe_cols = jnp.dot(texc, se, preferred_element_type=jnp.float32,
                        precision=lax.Precision.HIGHEST)
    offe_ref[...] = jnp.zeros((8, BLK), jnp.float32)
    offe_ref[0:1, 0:NBLK] = jnp.transpose(offe_cols)


def _prefix_stage(emb, w, b):
    return pl.pallas_call(
        _prefix_body,
        out_shape=(
            jax.ShapeDtypeStruct((SPAD, HID), jnp.float32),
            jax.ShapeDtypeStruct((OPAD, BLK), jnp.float32),
            jax.ShapeDtypeStruct((8, BLK), jnp.float32),
        ),
    )(emb, w, b)


def _span_body(lix_hbm, lie_hbm, offe_hbm, emb_hbm, spans_hbm,
               out_hbm, spans_v, lie_v, offe_v,
               ge0, gs0, ems0, eme0, ge1, gs1, ems1, eme1,
               ge2, gs2, ems2, eme2,
               ei0, si0, pi0, ei1, si1, pi1, ei2, si2, pi2,
               gsem0, gsem1, gsem2, osem0, osem1, osem2):
    ge = [ge0, ge1, ge2]
    gs = [gs0, gs1, gs2]
    ems = [ems0, ems1, ems2]
    eme = [eme0, eme1, eme2]
    eidx = [ei0, ei1, ei2]
    sidx = [si0, si1, si2]
    spidx = [pi0, pi1, pi2]
    gsem = [gsem0, gsem1, gsem2]
    osem = [osem0, osem1, osem2]

    wid = lax.axis_index("s") * NC + lax.axis_index("c")
    pltpu.sync_copy(spans_hbm.at[pl.ds(wid * SP_PER_W, SP_PER_W), :], spans_v)
    pltpu.sync_copy(lie_hbm, lie_v)
    pltpu.sync_copy(offe_hbm, offe_v)

    lane = lax.iota(jnp.int32, 16)
    zz = jnp.zeros((16,), jnp.int32)
    zo = jnp.ones((16,), jnp.int32)

    def span_vecs(t):
        gidx = jnp.minimum(t * GRP + lane, SP_PER_W - 1)
        s_vec = plsc.load_gather(spans_v, [gidx, zz])
        e_vec = plsc.load_gather(spans_v, [gidx, zo])
        s_is0 = s_vec == 0
        sp_vec = jnp.where(s_is0, SEQ, s_vec - 1)
        be_vec = lax.shift_right_logical(e_vec, 7)
        bsp_vec = jnp.where(s_is0, NBLK,
                            lax.shift_right_logical(s_vec - 1, 7))
        return s_vec, e_vec, sp_vec, be_vec, bsp_vec

    def issue_gathers(t, k):
        s_vec, e_vec, sp_vec, _, _ = span_vecs(t)
        eidx[k][...] = e_vec
        sidx[k][...] = s_vec
        spidx[k][...] = sp_vec
        ei = eidx[k].at[pl.ds(0, GRP)]
        si = sidx[k].at[pl.ds(0, GRP)]
        pi = spidx[k].at[pl.ds(0, GRP)]
        return [
            pltpu.async_copy(lix_hbm.at[ei], ge[k], gsem[k]),
            pltpu.async_copy(lix_hbm.at[pi], gs[k], gsem[k]),
            pltpu.async_copy(emb_hbm.at[si], ems[k], gsem[k]),
            pltpu.async_copy(emb_hbm.at[ei], eme[k], gsem[k]),
        ]

    pend_g = {0: issue_gathers(0, 0)}
    pend_o = {}

    for g in range(NGRP):
        k = g % NBUF
        kn = (g + 1) % NBUF
        if g + 1 < NGRP:
            for c in pend_o.pop(kn, ()):
                c.wait()
            pend_g[kn] = issue_gathers(g + 1, kn)
        for c in pend_g.pop(k):
            c.wait()

        _, e_vec, sp_vec, be_vec, bsp_vec = span_vecs(g)
        c127 = jnp.full((16,), 127, jnp.int32)
        den = (plsc.load_gather(offe_v, [zz, be_vec])
               + plsc.load_gather(lie_v, [lax.shift_right_logical(e_vec, 7),
                                          e_vec & c127])
               - plsc.load_gather(offe_v, [zz, bsp_vec])
               - plsc.load_gather(lie_v, [lax.shift_right_logical(sp_vec, 7),
                                          sp_vec & c127]))
        inv_vec = 1.0 / den
        inv = [inv_vec[j] for j in range(GRP)]

        gek, gsk = ge[k], gs[k]

        def chunk(c, carry):
            for u in range(2):
                o = c * 32 + u * 16
                for j in range(GRP):
                    num = gek[j, pl.ds(o, 16)] - gsk[j, pl.ds(o, 16)]
                    gsk[j, pl.ds(o, 16)] = num * inv[j]
            return carry

        lax.fori_loop(0, NCH // 2, chunk, 0)

        base = wid * SP_PER_W + g * GRP
        pend_o[k] = [
            pltpu.async_copy(ems[k],
                             out_hbm.at[pl.ds(base, GRP), pl.ds(0, HID)],
                             osem[k]),
            pltpu.async_copy(eme[k],
                             out_hbm.at[pl.ds(base, GRP), pl.ds(HID, HID)],
                             osem[k]),
            pltpu.async_copy(gs[k],
                             out_hbm.at[pl.ds(base, GRP), pl.ds(2 * HID, HID)],
                             osem[k]),
        ]

    for k in list(pend_o):
        for c in pend_o.pop(k):
            c.wait()


@functools.cache
def _make_span_stage():
    row_bufs = []
    for _ in range(NBUF):
        row_bufs += [pltpu.VMEM((GRP, HID), jnp.float32)] * 4
    idx_bufs = [pltpu.VMEM((16,), jnp.int32)] * (3 * NBUF)
    sems = [pltpu.SemaphoreType.DMA] * (2 * NBUF)
    return functools.partial(
        pl.kernel,
        out_type=jax.ShapeDtypeStruct((NSPANS, 3 * HID), jnp.float32),
        mesh=plsc.VectorSubcoreMesh(core_axis_name="c", subcore_axis_name="s"),
        compiler_params=pltpu.CompilerParams(needs_layout_passes=False),
        scratch_types=[
            pltpu.VMEM((SP_PER_W, 2), jnp.int32),
            pltpu.VMEM((OPAD, BLK), jnp.float32),
            pltpu.VMEM((8, BLK), jnp.float32),
        ] + row_bufs + idx_bufs + sems,
    )(_span_body)


@jax.jit
def kernel(embeddings, all_spans, W, b):
    emb = embeddings[0]
    w2 = W.reshape(1, HID)
    b2 = b.reshape(1, 1)
    lix, lie, offe = _prefix_stage(emb, w2, b2)
    return _make_span_stage()(lix, lie, offe, emb,
                              all_spans.astype(jnp.int32))
